# Initial kernel scaffold; baseline (speedup 1.0000x reference)
import jax
import jax.numpy as jnp
from jax.experimental import pallas as pl


def kernel(segment_embeddings, segment_mask, width_table, W1, b1, w2, b2):
    raise NotImplementedError("write your pallas kernel here")


# speedup vs baseline: 7.1804x; 7.1804x over previous
import functools
import math

import jax
import jax.numpy as jnp
from jax.experimental import pallas as pl

HID = 256
MAXW = 16
WF = 20
SPAN_DIM = 3 * HID + WF


def _gelu(x):
    return 0.5 * x * (1.0 + jax.lax.erf(x * (1.0 / math.sqrt(2.0))))


def _score_kernel(seq_ref, seg_ref, wt_ref, w1_ref, b1_ref, w2_ref, b2_ref,
                  out_ref):
    n = seg_ref.shape[1]
    x = seg_ref[0]
    a = w1_ref[:, 0:HID]
    bm = w1_ref[:, HID:2 * HID]
    c = w1_ref[:, 2 * HID:3 * HID]
    d = w1_ref[:, 3 * HID:SPAN_DIM]
    dims = (((1,), (1,)), ((), ()))
    xa = jax.lax.dot_general(x, a, dims, preferred_element_type=jnp.float32)
    yb = jax.lax.dot_general(x, bm, dims, preferred_element_type=jnp.float32)
    wd = jax.lax.dot_general(wt_ref[...], d, dims,
                             preferred_element_type=jnp.float32) + b1_ref[...]
    seq_len = seq_ref[0, 0, 0]
    b2 = b2_ref[0, 0]
    w2p = jnp.concatenate(
        [w2_ref[...], jnp.zeros((7, HID), jnp.float32)], axis=0)
    s_iota = jax.lax.broadcasted_iota(jnp.int32, (n, 1), 0).astype(jnp.float32)
    cols = [None] * MAXW
    for w in range(MAXW):
        xw = x if w == 0 else jnp.roll(x, -w, axis=0)
        ybw = yb if w == 0 else jnp.roll(yb, -w, axis=0)
        p = x * xw
        hpre = xa + ybw + wd[w:w + 1, :] + jax.lax.dot_general(
            p, c, dims, preferred_element_type=jnp.float32)
        h = _gelu(hpre)
        sw = jax.lax.dot_general(h, w2p, dims,
                                 preferred_element_type=jnp.float32)[:, 0:1] + b2
        e = s_iota + float(w)
        sw = sw + jnp.where(e >= seq_len, -1e6, 0.0)
        cols[MAXW - 1 - w] = jnp.where(e >= float(n), -1e30, sw)
    out_ref[0] = jnp.concatenate(cols, axis=1)


def _gather_kernel(seq_ref, idx_ref, seg_ref, wt_ref, w1_ref, b1_ref, w2_ref,
                   b2_ref, emb_ref, sc_ref):
    n = seg_ref.shape[1]
    x = seg_ref[0]
    idx = idx_ref[0]
    s_row = idx // MAXW
    j_row = idx % MAXW
    e_row = s_row + (MAXW - 1) - j_row
    w_row = (MAXW - 1) - j_row
    k = idx.shape[1]
    iota_n = jax.lax.broadcasted_iota(jnp.int32, (n, 1), 0)
    iota_w = jax.lax.broadcasted_iota(jnp.int32, (MAXW, 1), 0)
    oh_s = (iota_n == s_row).astype(jnp.float32)
    oh_e = (iota_n == e_row).astype(jnp.float32)
    oh_w = (iota_w == w_row).astype(jnp.float32)
    cdims = (((0,), (0,)), ((), ()))
    xg = jax.lax.dot_general(oh_s, x, cdims, preferred_element_type=jnp.float32)
    yg = jax.lax.dot_general(oh_e, x, cdims, preferred_element_type=jnp.float32)
    wg = jax.lax.dot_general(oh_w, wt_ref[...], cdims,
                             preferred_element_type=jnp.float32)
    a = w1_ref[:, 0:HID]
    bm = w1_ref[:, HID:2 * HID]
    c = w1_ref[:, 2 * HID:3 * HID]
    d = w1_ref[:, 3 * HID:SPAN_DIM]
    dims = (((1,), (1,)), ((), ()))
    hpre = (jax.lax.dot_general(xg, a, dims, preferred_element_type=jnp.float32)
            + jax.lax.dot_general(yg, bm, dims,
                                  preferred_element_type=jnp.float32)
            + jax.lax.dot_general(xg * yg, c, dims,
                                  preferred_element_type=jnp.float32)
            + jax.lax.dot_general(wg, d, dims,
                                  preferred_element_type=jnp.float32)
            + b1_ref[...])
    h = _gelu(hpre)
    emb_ref[0] = h
    w2p = jnp.concatenate(
        [w2_ref[...], jnp.zeros((7, HID), jnp.float32)], axis=0)
    sc = jax.lax.dot_general(w2p, h, dims,
                             preferred_element_type=jnp.float32)[0:1]
    seq_len = seq_ref[0, 0, 0]
    pen = jnp.where(e_row.astype(jnp.float32) >= seq_len, -1e6, 0.0)
    sc_ref[0] = sc + b2_ref[0, 0] + pen


def kernel(segment_embeddings, segment_mask, width_table, W1, b1, w2, b2):
    B, N, H = segment_embeddings.shape
    k = int(N * 0.4)
    kpad = ((k + 127) // 128) * 128
    seq_lens = jnp.sum(segment_mask, axis=-1).reshape(B, 1, 1)
    b1r = b1.reshape(1, H)
    b2r = b2.reshape(1, 1)

    scores = pl.pallas_call(
        _score_kernel,
        grid=(B,),
        in_specs=[
            pl.BlockSpec((1, 1, 1), lambda b: (b, 0, 0)),
            pl.BlockSpec((1, N, H), lambda b: (b, 0, 0)),
            pl.BlockSpec((MAXW, WF), lambda b: (0, 0)),
            pl.BlockSpec((H, SPAN_DIM), lambda b: (0, 0)),
            pl.BlockSpec((1, H), lambda b: (0, 0)),
            pl.BlockSpec((1, H), lambda b: (0, 0)),
            pl.BlockSpec((1, 1), lambda b: (0, 0)),
        ],
        out_specs=pl.BlockSpec((1, N, MAXW), lambda b: (b, 0, 0)),
        out_shape=jax.ShapeDtypeStruct((B, N, MAXW), jnp.float32),
    )(seq_lens, segment_embeddings, width_table, W1, b1r, w2, b2r)

    flat = scores.reshape(B, N * MAXW)
    _, top_idx = jax.lax.top_k(flat, k)
    top_idx = jnp.sort(top_idx, axis=1)
    idx3 = jnp.pad(top_idx, ((0, 0), (0, kpad - k))).reshape(B, 1, kpad)

    embs, sc3 = pl.pallas_call(
        _gather_kernel,
        grid=(B,),
        in_specs=[
            pl.BlockSpec((1, 1, 1), lambda b: (b, 0, 0)),
            pl.BlockSpec((1, 1, kpad), lambda b: (b, 0, 0)),
            pl.BlockSpec((1, N, H), lambda b: (b, 0, 0)),
            pl.BlockSpec((MAXW, WF), lambda b: (0, 0)),
            pl.BlockSpec((H, SPAN_DIM), lambda b: (0, 0)),
            pl.BlockSpec((1, H), lambda b: (0, 0)),
            pl.BlockSpec((1, H), lambda b: (0, 0)),
            pl.BlockSpec((1, 1), lambda b: (0, 0)),
        ],
        out_specs=[
            pl.BlockSpec((1, kpad, H), lambda b: (b, 0, 0)),
            pl.BlockSpec((1, 1, kpad), lambda b: (b, 0, 0)),
        ],
        out_shape=[
            jax.ShapeDtypeStruct((B, kpad, H), jnp.float32),
            jax.ShapeDtypeStruct((B, 1, kpad), jnp.float32),
        ],
    )(seq_lens, idx3, segment_embeddings, width_table, W1, b1r, w2, b2r)

    top_embs = embs[:, :k, :]
    top_scores = sc3[:, 0, :k]
    s = top_idx // MAXW
    e = s + (MAXW - 1) - (top_idx % MAXW)
    top_spans = jnp.stack([s, e], axis=-1)
    return top_embs, top_scores, top_spans


# speedup vs baseline: 12.7612x; 1.7772x over previous
import math

import jax
import jax.numpy as jnp
from jax.experimental import pallas as pl

HID = 256
MAXW = 16
WF = 20
SPAN_DIM = 3 * HID + WF
INT_MIN = -2147483648
INT_MAX = 2147483647


def _gelu(x):
    return 0.5 * x * (1.0 + jax.lax.erf(x * (1.0 / math.sqrt(2.0))))


def _score_kernel(seq_ref, seg_ref, wt_ref, w1_ref, b1_ref, w2_ref, b2_ref,
                  out_ref):
    n = seg_ref.shape[1]
    x = seg_ref[0]
    a = w1_ref[:, 0:HID]
    bm = w1_ref[:, HID:2 * HID]
    c = w1_ref[:, 2 * HID:3 * HID]
    d = w1_ref[:, 3 * HID:SPAN_DIM]
    dims = (((1,), (1,)), ((), ()))
    xa = jax.lax.dot_general(x, a, dims, preferred_element_type=jnp.float32)
    yb = jax.lax.dot_general(x, bm, dims, preferred_element_type=jnp.float32)
    wd = jax.lax.dot_general(wt_ref[...], d, dims,
                             preferred_element_type=jnp.float32) + b1_ref[...]
    seq_len = seq_ref[0, 0, 0]
    b2 = b2_ref[0, 0]
    w2p = jnp.concatenate(
        [w2_ref[...], jnp.zeros((7, HID), jnp.float32)], axis=0)
    s_iota = jax.lax.broadcasted_iota(jnp.int32, (n, 1), 0).astype(jnp.float32)
    cols = [None] * MAXW
    for w in range(MAXW):
        xw = x if w == 0 else jnp.roll(x, -w, axis=0)
        ybw = yb if w == 0 else jnp.roll(yb, -w, axis=0)
        p = x * xw
        hpre = xa + ybw + wd[w:w + 1, :] + jax.lax.dot_general(
            p, c, dims, preferred_element_type=jnp.float32)
        h = _gelu(hpre)
        sw = jax.lax.dot_general(h, w2p, dims,
                                 preferred_element_type=jnp.float32)[:, 0:1] + b2
        e = s_iota + float(w)
        sw = sw + jnp.where(e >= seq_len, -1e6, 0.0)
        cols[MAXW - 1 - w] = jnp.where(e >= float(n), -1e30, sw)
    out_ref[0] = jnp.concatenate(cols, axis=1)


def _to_key(x):
    x = jnp.where(x == 0.0, 0.0, x)
    b = jax.lax.bitcast_convert_type(x, jnp.int32)
    return jnp.where(b >= 0, b,
                     jnp.bitwise_xor(jnp.bitwise_not(b), jnp.int32(INT_MIN)))


def _make_select_kernel(nbatch, n, k, kpad):
    def _select_kernel(sc16_ref, sc128_ref, sef_ref):
        keys_c = _to_key(sc128_ref[...])

        def count_ge(mid):
            ge = (keys_c >= mid).astype(jnp.int32)
            return jnp.sum(jnp.sum(ge, axis=2, keepdims=True), axis=1,
                           keepdims=True)

        zero = jnp.zeros((nbatch, 1, 1), jnp.int32)
        c0 = count_ge(zero)
        lo = jnp.where(c0 >= k, 0, INT_MIN)
        hi = jnp.where(c0 >= k, INT_MAX, -1)

        def body(_, carry):
            lo, hi = carry
            d = hi - lo
            mid = lo + jax.lax.shift_right_arithmetic(d, 1) + (d & 1)
            ge = count_ge(mid) >= k
            return jnp.where(ge, mid, lo), jnp.where(ge, hi, mid - 1)

        lo, hi = jax.lax.fori_loop(0, 31, body, (lo, hi))
        t = lo
        m = jnp.sum(jnp.sum((keys_c > t).astype(jnp.int32), axis=2,
                            keepdims=True), axis=1, keepdims=True)
        need = (k - m).astype(jnp.float32)

        key16 = _to_key(sc16_ref[...])
        iota_j = jax.lax.broadcasted_iota(jnp.int32, (MAXW, MAXW), 0)
        iota_j2 = jax.lax.broadcasted_iota(jnp.int32, (MAXW, MAXW), 1)
        u16 = (iota_j <= iota_j2).astype(jnp.float32)
        iota_a = jax.lax.broadcasted_iota(jnp.int32, (n, n), 0)
        iota_b = jax.lax.broadcasted_iota(jnp.int32, (n, n), 1)
        lst = (iota_b < iota_a).astype(jnp.float32)
        ust = (iota_a < iota_b).astype(jnp.float32)
        ones8 = jnp.ones((8, MAXW), jnp.float32)
        pcol = jax.lax.broadcasted_iota(jnp.int32, (kpad, 1), 0).astype(
            jnp.float32)
        iota_row = jax.lax.broadcasted_iota(jnp.int32, (1, n), 1).astype(
            jnp.float32)
        cd10 = (((1,), (0,)), ((), ()))
        cd11 = (((1,), (1,)), ((), ()))
        for b in range(nbatch):
            kb = key16[b]
            tb = t[b]
            gt = (kb > tb).astype(jnp.float32)
            eq = (kb == tb).astype(jnp.float32)
            cumc_eq = jax.lax.dot_general(eq, u16, cd10,
                                          preferred_element_type=jnp.float32)
            req = jnp.sum(eq, axis=1, keepdims=True)
            req8 = jnp.concatenate([req, jnp.zeros((n, 7), jnp.float32)], 1)
            peq = jax.lax.dot_general(lst, req8, cd10,
                                      preferred_element_type=jnp.float32)[:, 0:1]
            eqrank = cumc_eq + peq
            sel = gt + eq * (eqrank <= need[b]).astype(jnp.float32)
            cumc = jax.lax.dot_general(sel, u16, cd10,
                                       preferred_element_type=jnp.float32)
            rrow = jax.lax.dot_general(ones8, sel, cd11,
                                       preferred_element_type=jnp.float32)
            prow = jax.lax.dot_general(rrow, ust, cd10,
                                       preferred_element_type=jnp.float32)[0:1]
            le = (prow <= pcol).astype(jnp.float32)
            rp = jnp.sum(le, axis=1, keepdims=True) - 1.0
            ohr = (rp == iota_row).astype(jnp.float32)
            cumrow = jax.lax.dot_general(ohr, cumc, cd10,
                                         preferred_element_type=jnp.float32)
            prp = jnp.sum(ohr * prow, axis=1, keepdims=True)
            tp = pcol + 1.0 - prp
            cp = jnp.sum((cumrow < tp).astype(jnp.float32), axis=1,
                         keepdims=True)
            scol = rp
            ecol = rp + float(MAXW - 1) - cp
            fcol = rp * float(MAXW) + cp
            sef_ref[b] = jnp.concatenate(
                [scol, ecol, fcol, jnp.zeros((kpad, 5), jnp.float32)], axis=1)
    return _select_kernel


def _gather_kernel(seq_ref, sef_ref, seg_ref, wt_ref, w1_ref, b1_ref, w2_ref,
                   b2_ref, emb_ref, sc_ref):
    n = seg_ref.shape[1]
    x = seg_ref[0]
    s_col = sef_ref[0][:, 0:1]
    e_col = sef_ref[0][:, 1:2]
    w_col = e_col - s_col
    iota_n = jax.lax.broadcasted_iota(jnp.int32, (1, n), 1).astype(jnp.float32)
    iota_w = jax.lax.broadcasted_iota(jnp.int32, (1, MAXW), 1).astype(
        jnp.float32)
    oh_s = (s_col == iota_n).astype(jnp.float32)
    oh_e = (e_col == iota_n).astype(jnp.float32)
    oh_w = (w_col == iota_w).astype(jnp.float32)
    cd10 = (((1,), (0,)), ((), ()))
    xg = jax.lax.dot_general(oh_s, x, cd10, preferred_element_type=jnp.float32)
    yg = jax.lax.dot_general(oh_e, x, cd10, preferred_element_type=jnp.float32)
    wg = jax.lax.dot_general(oh_w, wt_ref[...], cd10,
                             preferred_element_type=jnp.float32)
    a = w1_ref[:, 0:HID]
    bm = w1_ref[:, HID:2 * HID]
    c = w1_ref[:, 2 * HID:3 * HID]
    d = w1_ref[:, 3 * HID:SPAN_DIM]
    dims = (((1,), (1,)), ((), ()))
    hpre = (jax.lax.dot_general(xg, a, dims, preferred_element_type=jnp.float32)
            + jax.lax.dot_general(yg, bm, dims,
                                  preferred_element_type=jnp.float32)
            + jax.lax.dot_general(xg * yg, c, dims,
                                  preferred_element_type=jnp.float32)
            + jax.lax.dot_general(wg, d, dims,
                                  preferred_element_type=jnp.float32)
            + b1_ref[...])
    h = _gelu(hpre)
    emb_ref[0] = h
    w2p = jnp.concatenate(
        [w2_ref[...], jnp.zeros((7, HID), jnp.float32)], axis=0)
    sc = jax.lax.dot_general(h, w2p, dims,
                             preferred_element_type=jnp.float32)
    seq_len = seq_ref[0, 0, 0]
    pen = jnp.where(e_col >= seq_len, -1e6, 0.0)
    sc_ref[0] = sc + b2_ref[0, 0] + pen


def kernel(segment_embeddings, segment_mask, width_table, W1, b1, w2, b2):
    B, N, H = segment_embeddings.shape
    k = int(N * 0.4)
    kpad = ((k + 127) // 128) * 128
    seq_lens = jnp.sum(segment_mask, axis=-1).reshape(B, 1, 1)
    b1r = b1.reshape(1, H)
    b2r = b2.reshape(1, 1)

    scores = pl.pallas_call(
        _score_kernel,
        grid=(B,),
        in_specs=[
            pl.BlockSpec((1, 1, 1), lambda b: (b, 0, 0)),
            pl.BlockSpec((1, N, H), lambda b: (b, 0, 0)),
            pl.BlockSpec((MAXW, WF), lambda b: (0, 0)),
            pl.BlockSpec((H, SPAN_DIM), lambda b: (0, 0)),
            pl.BlockSpec((1, H), lambda b: (0, 0)),
            pl.BlockSpec((1, H), lambda b: (0, 0)),
            pl.BlockSpec((1, 1), lambda b: (0, 0)),
        ],
        out_specs=pl.BlockSpec((1, N, MAXW), lambda b: (b, 0, 0)),
        out_shape=jax.ShapeDtypeStruct((B, N, MAXW), jnp.float32),
    )(seq_lens, segment_embeddings, width_table, W1, b1r, w2, b2r)

    sc128 = scores.reshape(B, 128, (N * MAXW) // 128)

    sef = pl.pallas_call(
        _make_select_kernel(B, N, k, kpad),
        grid=(1,),
        in_specs=[
            pl.BlockSpec((B, N, MAXW), lambda i: (0, 0, 0)),
            pl.BlockSpec(sc128.shape, lambda i: (0, 0, 0)),
        ],
        out_specs=pl.BlockSpec((B, kpad, 8), lambda i: (0, 0, 0)),
        out_shape=jax.ShapeDtypeStruct((B, kpad, 8), jnp.float32),
    )(scores, sc128)

    embs, scc = pl.pallas_call(
        _gather_kernel,
        grid=(B,),
        in_specs=[
            pl.BlockSpec((1, 1, 1), lambda b: (b, 0, 0)),
            pl.BlockSpec((1, kpad, 8), lambda b: (b, 0, 0)),
            pl.BlockSpec((1, N, H), lambda b: (b, 0, 0)),
            pl.BlockSpec((MAXW, WF), lambda b: (0, 0)),
            pl.BlockSpec((H, SPAN_DIM), lambda b: (0, 0)),
            pl.BlockSpec((1, H), lambda b: (0, 0)),
            pl.BlockSpec((1, H), lambda b: (0, 0)),
            pl.BlockSpec((1, 1), lambda b: (0, 0)),
        ],
        out_specs=[
            pl.BlockSpec((1, kpad, H), lambda b: (b, 0, 0)),
            pl.BlockSpec((1, kpad, 8), lambda b: (b, 0, 0)),
        ],
        out_shape=[
            jax.ShapeDtypeStruct((B, kpad, H), jnp.float32),
            jax.ShapeDtypeStruct((B, kpad, 8), jnp.float32),
        ],
    )(seq_lens, sef, segment_embeddings, width_table, W1, b1r, w2, b2r)

    top_embs = embs[:, :k, :]
    top_scores = scc[:, :k, 0]
    top_spans = sef[:, :k, 0:2].astype(jnp.int32)
    return top_embs, top_scores, top_spans


# speedup vs baseline: 12.8567x; 1.0075x over previous
import math

import jax
import jax.numpy as jnp
from jax.experimental import pallas as pl

HID = 256
MAXW = 16
WF = 20
SPAN_DIM = 3 * HID + WF
INT_MIN = -2147483648
INT_MAX = 2147483647


def _gelu(x):
    return 0.5 * x * (1.0 + jax.lax.erf(x * (1.0 / math.sqrt(2.0))))


def _score_kernel(seq_ref, seg_ref, wt_ref, w1_ref, b1_ref, w2_ref, b2_ref,
                  out_ref):
    n = seg_ref.shape[1]
    x = seg_ref[0]
    a = w1_ref[:, 0:HID]
    bm = w1_ref[:, HID:2 * HID]
    c = w1_ref[:, 2 * HID:3 * HID]
    d = w1_ref[:, 3 * HID:SPAN_DIM]
    dims = (((1,), (1,)), ((), ()))
    xa = jax.lax.dot_general(x, a, dims, preferred_element_type=jnp.float32)
    yb = jax.lax.dot_general(x, bm, dims, preferred_element_type=jnp.float32)
    wd = jax.lax.dot_general(wt_ref[...], d, dims,
                             preferred_element_type=jnp.float32) + b1_ref[...]
    seq_len = seq_ref[0, 0, 0]
    b2 = b2_ref[0, 0]
    w2p = jnp.concatenate(
        [w2_ref[...], jnp.zeros((7, HID), jnp.float32)], axis=0)
    s_iota = jax.lax.broadcasted_iota(jnp.int32, (n, 1), 0).astype(jnp.float32)
    cols = [None] * MAXW
    for w in range(MAXW):
        xw = x if w == 0 else jnp.roll(x, -w, axis=0)
        ybw = yb if w == 0 else jnp.roll(yb, -w, axis=0)
        p = x * xw
        hpre = xa + ybw + wd[w:w + 1, :] + jax.lax.dot_general(
            p, c, dims, preferred_element_type=jnp.float32)
        h = _gelu(hpre)
        sw = jax.lax.dot_general(h, w2p, dims,
                                 preferred_element_type=jnp.float32)[:, 0:1] + b2
        e = s_iota + float(w)
        sw = sw + jnp.where(e >= seq_len, -1e6, 0.0)
        cols[MAXW - 1 - w] = jnp.where(e >= float(n), -1e30, sw)
    out_ref[0] = jnp.concatenate(cols, axis=1)


def _to_key(x):
    x = jnp.where(x == 0.0, 0.0, x)
    b = jax.lax.bitcast_convert_type(x, jnp.int32)
    return jnp.where(b >= 0, b,
                     jnp.bitwise_xor(jnp.bitwise_not(b), jnp.int32(INT_MIN)))


def _make_select_gather_kernel(nbatch, n, k, kpad):
    def _select_kernel(sc16_ref, sc128_ref, seq_ref, seg_ref, wt_ref, w1_ref,
                       b1_ref, w2_ref, b2_ref, sef_ref, emb_ref, scc_ref):
        keys_c = _to_key(sc128_ref[...])

        def count_ge(mid):
            ge = (keys_c >= mid).astype(jnp.int32)
            return jnp.sum(jnp.sum(ge, axis=2, keepdims=True), axis=1,
                           keepdims=True)

        zero = jnp.zeros((nbatch, 1, 1), jnp.int32)
        c0 = count_ge(zero)
        lo = jnp.where(c0 >= k, 0, INT_MIN)
        hi = jnp.where(c0 >= k, INT_MAX, -1)

        def body4(_, carry):
            lo, hi = carry
            d = hi - lo
            d2 = jax.lax.shift_right_arithmetic(d, 2)
            r = d & 3
            m1 = lo + d2 + (r > 0).astype(jnp.int32)
            m2 = lo + 2 * d2 + jax.lax.shift_right_arithmetic(r + 1, 1)
            m3 = lo + 3 * d2 + jax.lax.shift_right_arithmetic(3 * r + 3, 2)
            g1 = count_ge(m1) >= k
            g2 = count_ge(m2) >= k
            g3 = count_ge(m3) >= k
            nlo = jnp.where(g3, m3, jnp.where(g2, m2, jnp.where(g1, m1, lo)))
            nhi = jnp.where(g3, hi,
                            jnp.where(g2, m3 - 1,
                                      jnp.where(g1, m2 - 1, m1 - 1)))
            return nlo, nhi

        def body2(_, carry):
            lo, hi = carry
            d = hi - lo
            mid = lo + jax.lax.shift_right_arithmetic(d, 1) + (d & 1)
            ge = count_ge(mid) >= k
            return jnp.where(ge, mid, lo), jnp.where(ge, hi, mid - 1)

        lo, hi = jax.lax.fori_loop(0, 16, body4, (lo, hi))
        lo, hi = jax.lax.fori_loop(0, 2, body2, (lo, hi))
        t = lo
        m = jnp.sum(jnp.sum((keys_c > t).astype(jnp.int32), axis=2,
                            keepdims=True), axis=1, keepdims=True)
        need = (k - m).astype(jnp.float32)

        key16 = _to_key(sc16_ref[...])
        iota_j = jax.lax.broadcasted_iota(jnp.int32, (MAXW, MAXW), 0)
        iota_j2 = jax.lax.broadcasted_iota(jnp.int32, (MAXW, MAXW), 1)
        u16 = (iota_j <= iota_j2).astype(jnp.float32)
        iota_a = jax.lax.broadcasted_iota(jnp.int32, (n, n), 0)
        iota_b = jax.lax.broadcasted_iota(jnp.int32, (n, n), 1)
        lst = (iota_b < iota_a).astype(jnp.float32)
        ust = (iota_a < iota_b).astype(jnp.float32)
        ones8 = jnp.ones((8, MAXW), jnp.float32)
        pcol = jax.lax.broadcasted_iota(jnp.int32, (kpad, 1), 0).astype(
            jnp.float32)
        iota_row = jax.lax.broadcasted_iota(jnp.int32, (1, n), 1).astype(
            jnp.float32)
        cd10 = (((1,), (0,)), ((), ()))
        cd11 = (((1,), (1,)), ((), ()))
        dims = (((1,), (1,)), ((), ()))
        iota_wr = jax.lax.broadcasted_iota(jnp.int32, (1, MAXW), 1).astype(
            jnp.float32)
        aw = w1_ref[:, 0:HID]
        bw = w1_ref[:, HID:2 * HID]
        cw = w1_ref[:, 2 * HID:3 * HID]
        dw = w1_ref[:, 3 * HID:SPAN_DIM]
        w2p = jnp.concatenate(
            [w2_ref[...], jnp.zeros((7, HID), jnp.float32)], axis=0)
        for b in range(nbatch):
            kb = key16[b]
            tb = t[b]
            gt = (kb > tb).astype(jnp.float32)
            eq = (kb == tb).astype(jnp.float32)
            cumc_eq = jax.lax.dot_general(eq, u16, cd10,
                                          preferred_element_type=jnp.float32)
            req = jnp.sum(eq, axis=1, keepdims=True)
            req8 = jnp.concatenate([req, jnp.zeros((n, 7), jnp.float32)], 1)
            peq = jax.lax.dot_general(lst, req8, cd10,
                                      preferred_element_type=jnp.float32)[:, 0:1]
            eqrank = cumc_eq + peq
            sel = gt + eq * (eqrank <= need[b]).astype(jnp.float32)
            cumc = jax.lax.dot_general(sel, u16, cd10,
                                       preferred_element_type=jnp.float32)
            rrow = jax.lax.dot_general(ones8, sel, cd11,
                                       preferred_element_type=jnp.float32)
            prow = jax.lax.dot_general(rrow, ust, cd10,
                                       preferred_element_type=jnp.float32)[0:1]
            le = (prow <= pcol).astype(jnp.float32)
            rp = jnp.sum(le, axis=1, keepdims=True) - 1.0
            ohr = (rp == iota_row).astype(jnp.float32)
            cumrow = jax.lax.dot_general(ohr, cumc, cd10,
                                         preferred_element_type=jnp.float32)
            prp = jnp.sum(ohr * prow, axis=1, keepdims=True)
            tp = pcol + 1.0 - prp
            cp = jnp.sum((cumrow < tp).astype(jnp.float32), axis=1,
                         keepdims=True)
            scol = rp
            ecol = rp + float(MAXW - 1) - cp
            fcol = rp * float(MAXW) + cp
            sef_ref[b] = jnp.concatenate(
                [scol, ecol, fcol, jnp.zeros((kpad, 5), jnp.float32)], axis=1)
            x = seg_ref[b]
            w_col = ecol - scol
            oh_s = (scol == iota_row).astype(jnp.float32)
            oh_e = (ecol == iota_row).astype(jnp.float32)
            oh_w = (w_col == iota_wr).astype(jnp.float32)
            xg = jax.lax.dot_general(oh_s, x, cd10,
                                     preferred_element_type=jnp.float32)
            yg = jax.lax.dot_general(oh_e, x, cd10,
                                     preferred_element_type=jnp.float32)
            wg = jax.lax.dot_general(oh_w, wt_ref[...], cd10,
                                     preferred_element_type=jnp.float32)
            hpre = (jax.lax.dot_general(xg, aw, dims,
                                        preferred_element_type=jnp.float32)
                    + jax.lax.dot_general(yg, bw, dims,
                                          preferred_element_type=jnp.float32)
                    + jax.lax.dot_general(xg * yg, cw, dims,
                                          preferred_element_type=jnp.float32)
                    + jax.lax.dot_general(wg, dw, dims,
                                          preferred_element_type=jnp.float32)
                    + b1_ref[...])
            h = _gelu(hpre)
            emb_ref[b] = h
            sc = jax.lax.dot_general(h, w2p, dims,
                                     preferred_element_type=jnp.float32)
            pen = jnp.where(ecol >= seq_ref[b], -1e6, 0.0)
            scc_ref[b] = sc + b2_ref[0, 0] + pen
    return _select_kernel


def kernel(segment_embeddings, segment_mask, width_table, W1, b1, w2, b2):
    B, N, H = segment_embeddings.shape
    k = int(N * 0.4)
    kpad = ((k + 7) // 8) * 8
    seq_lens = jnp.sum(segment_mask, axis=-1).reshape(B, 1, 1)
    b1r = b1.reshape(1, H)
    b2r = b2.reshape(1, 1)

    scores = pl.pallas_call(
        _score_kernel,
        grid=(B,),
        in_specs=[
            pl.BlockSpec((1, 1, 1), lambda b: (b, 0, 0)),
            pl.BlockSpec((1, N, H), lambda b: (b, 0, 0)),
            pl.BlockSpec((MAXW, WF), lambda b: (0, 0)),
            pl.BlockSpec((H, SPAN_DIM), lambda b: (0, 0)),
            pl.BlockSpec((1, H), lambda b: (0, 0)),
            pl.BlockSpec((1, H), lambda b: (0, 0)),
            pl.BlockSpec((1, 1), lambda b: (0, 0)),
        ],
        out_specs=pl.BlockSpec((1, N, MAXW), lambda b: (b, 0, 0)),
        out_shape=jax.ShapeDtypeStruct((B, N, MAXW), jnp.float32),
    )(seq_lens, segment_embeddings, width_table, W1, b1r, w2, b2r)

    sc128 = scores.reshape(B, 128, (N * MAXW) // 128)

    sef, embs, scc = pl.pallas_call(
        _make_select_gather_kernel(B, N, k, kpad),
        grid=(1,),
        in_specs=[
            pl.BlockSpec((B, N, MAXW), lambda i: (0, 0, 0)),
            pl.BlockSpec((B, 128, (N * MAXW) // 128), lambda i: (0, 0, 0)),
            pl.BlockSpec((B, 1, 1), lambda i: (0, 0, 0)),
            pl.BlockSpec((B, N, H), lambda i: (0, 0, 0)),
            pl.BlockSpec((MAXW, WF), lambda i: (0, 0)),
            pl.BlockSpec((H, SPAN_DIM), lambda i: (0, 0)),
            pl.BlockSpec((1, H), lambda i: (0, 0)),
            pl.BlockSpec((1, H), lambda i: (0, 0)),
            pl.BlockSpec((1, 1), lambda i: (0, 0)),
        ],
        out_specs=[
            pl.BlockSpec((B, kpad, 8), lambda i: (0, 0, 0)),
            pl.BlockSpec((B, kpad, H), lambda i: (0, 0, 0)),
            pl.BlockSpec((B, kpad, 8), lambda i: (0, 0, 0)),
        ],
        out_shape=[
            jax.ShapeDtypeStruct((B, kpad, 8), jnp.float32),
            jax.ShapeDtypeStruct((B, kpad, H), jnp.float32),
            jax.ShapeDtypeStruct((B, kpad, 8), jnp.float32),
        ],
    )(scores, sc128, seq_lens, segment_embeddings, width_table, W1, b1r, w2,
      b2r)

    top_embs = embs[:, :k, :]
    top_scores = scc[:, :k, 0]
    top_spans = sef[:, :k, 0:2].astype(jnp.int32)
    return top_embs, top_scores, top_spans


# speedup vs baseline: 13.5531x; 1.0542x over previous
import math

import jax
import jax.numpy as jnp
from jax.experimental import pallas as pl

HID = 256
MAXW = 16
WF = 20
SPAN_DIM = 3 * HID + WF
INT_MIN = -2147483648
INT_MAX = 2147483647


def _gelu(x):
    return 0.5 * x * (1.0 + jax.lax.erf(x * (1.0 / math.sqrt(2.0))))


def _score_kernel(seq_ref, seg_ref, wt_ref, w1_ref, b1_ref, w2_ref, b2_ref,
                  out_ref):
    n = seg_ref.shape[1]
    x = seg_ref[0]
    a = w1_ref[:, 0:HID]
    bm = w1_ref[:, HID:2 * HID]
    c = w1_ref[:, 2 * HID:3 * HID]
    d = w1_ref[:, 3 * HID:SPAN_DIM]
    dims = (((1,), (1,)), ((), ()))
    xa = jax.lax.dot_general(x, a, dims, preferred_element_type=jnp.float32)
    yb = jax.lax.dot_general(x, bm, dims, preferred_element_type=jnp.float32)
    wd = jax.lax.dot_general(wt_ref[...], d, dims,
                             preferred_element_type=jnp.float32) + b1_ref[...]
    seq_len = seq_ref[0, 0, 0]
    b2 = b2_ref[0, 0]
    w2p = jnp.concatenate(
        [w2_ref[...], jnp.zeros((7, HID), jnp.float32)], axis=0)
    s_iota = jax.lax.broadcasted_iota(jnp.int32, (n, 1), 0).astype(jnp.float32)
    cols = [None] * MAXW
    for w in range(MAXW):
        xw = x if w == 0 else jnp.roll(x, -w, axis=0)
        ybw = yb if w == 0 else jnp.roll(yb, -w, axis=0)
        p = x * xw
        hpre = xa + ybw + wd[w:w + 1, :] + jax.lax.dot_general(
            p, c, dims, preferred_element_type=jnp.float32)
        h = _gelu(hpre)
        sw = jax.lax.dot_general(h, w2p, dims,
                                 preferred_element_type=jnp.float32)[:, 0:1] + b2
        e = s_iota + float(w)
        sw = sw + jnp.where(e >= seq_len, -1e6, 0.0)
        cols[MAXW - 1 - w] = jnp.where(e >= float(n), -1e30, sw)
    mm = jnp.concatenate(cols, axis=1)
    cd10c = (((1,), (0,)), ((), ()))
    s_io = jax.lax.broadcasted_iota(jnp.int32, (1, n), 1)
    r_io = jax.lax.broadcasted_iota(jnp.int32, (128, 1), 0)
    j_io = jax.lax.broadcasted_iota(jnp.int32, (MAXW, 1), 0)
    c_io = jax.lax.broadcasted_iota(jnp.int32, (1, 128), 1)
    acc = jnp.zeros((128, 128), jnp.float32)
    for q in range(8):
        pq = (s_io == 8 * r_io + q).astype(jnp.float32)
        eqm = (c_io == MAXW * q + j_io).astype(jnp.float32)
        acc = acc + jax.lax.dot_general(
            jax.lax.dot_general(pq, mm, cd10c,
                                preferred_element_type=jnp.float32),
            eqm, cd10c, preferred_element_type=jnp.float32)
    out_ref[0] = acc


def _to_key(x):
    x = jnp.where(x == 0.0, 0.0, x)
    b = jax.lax.bitcast_convert_type(x, jnp.int32)
    return jnp.where(b >= 0, b,
                     jnp.bitwise_xor(jnp.bitwise_not(b), jnp.int32(INT_MIN)))


def _make_select_gather_kernel(nbatch, n, k, kpad):
    def _select_kernel(sc128_ref, seq_ref, seg_ref, wt_ref, w1_ref,
                       b1_ref, w2_ref, b2_ref, sef_ref, emb_ref, scc_ref):
        keys_c = _to_key(sc128_ref[...])

        def count_ge(mid):
            ge = (keys_c >= mid).astype(jnp.int32)
            return jnp.sum(jnp.sum(ge, axis=2, keepdims=True), axis=1,
                           keepdims=True)

        zero = jnp.zeros((nbatch, 1, 1), jnp.int32)
        c0 = count_ge(zero)
        lo = jnp.where(c0 >= k, 0, INT_MIN)
        hi = jnp.where(c0 >= k, INT_MAX, -1)

        def body4(_, carry):
            lo, hi = carry
            d = hi - lo
            d2 = jax.lax.shift_right_arithmetic(d, 2)
            r = d & 3
            m1 = lo + d2 + (r > 0).astype(jnp.int32)
            m2 = lo + 2 * d2 + jax.lax.shift_right_arithmetic(r + 1, 1)
            m3 = lo + 3 * d2 + jax.lax.shift_right_arithmetic(3 * r + 3, 2)
            g1 = count_ge(m1) >= k
            g2 = count_ge(m2) >= k
            g3 = count_ge(m3) >= k
            nlo = jnp.where(g3, m3, jnp.where(g2, m2, jnp.where(g1, m1, lo)))
            nhi = jnp.where(g3, hi,
                            jnp.where(g2, m3 - 1,
                                      jnp.where(g1, m2 - 1, m1 - 1)))
            return nlo, nhi

        def body2(_, carry):
            lo, hi = carry
            d = hi - lo
            mid = lo + jax.lax.shift_right_arithmetic(d, 1) + (d & 1)
            ge = count_ge(mid) >= k
            return jnp.where(ge, mid, lo), jnp.where(ge, hi, mid - 1)

        lo, hi = jax.lax.fori_loop(0, 16, body4, (lo, hi))
        lo, hi = jax.lax.fori_loop(0, 2, body2, (lo, hi))
        t = lo
        m = jnp.sum(jnp.sum((keys_c > t).astype(jnp.int32), axis=2,
                            keepdims=True), axis=1, keepdims=True)
        need = (k - m).astype(jnp.float32)

        nr = sc128_ref.shape[1]
        nc = sc128_ref.shape[2]
        iota_a = jax.lax.broadcasted_iota(jnp.int32, (nc, nc), 0)
        iota_b = jax.lax.broadcasted_iota(jnp.int32, (nc, nc), 1)
        ucum = (iota_a <= iota_b).astype(jnp.float32)
        lst = (iota_b < iota_a).astype(jnp.float32)
        ust = (iota_a < iota_b).astype(jnp.float32)
        ones8 = jnp.ones((8, nc), jnp.float32)
        pcol = jax.lax.broadcasted_iota(jnp.int32, (kpad, 1), 0).astype(
            jnp.float32)
        iota_rr = jax.lax.broadcasted_iota(jnp.int32, (1, nr), 1).astype(
            jnp.float32)
        iota_row = jax.lax.broadcasted_iota(jnp.int32, (1, n), 1).astype(
            jnp.float32)
        cd10 = (((1,), (0,)), ((), ()))
        cd11 = (((1,), (1,)), ((), ()))
        dims = (((1,), (1,)), ((), ()))
        iota_wr = jax.lax.broadcasted_iota(jnp.int32, (1, MAXW), 1).astype(
            jnp.float32)
        aw = w1_ref[:, 0:HID]
        bw = w1_ref[:, HID:2 * HID]
        cw = w1_ref[:, 2 * HID:3 * HID]
        dw = w1_ref[:, 3 * HID:SPAN_DIM]
        w2p = jnp.concatenate(
            [w2_ref[...], jnp.zeros((7, HID), jnp.float32)], axis=0)
        for b in range(nbatch):
            kb = keys_c[b]
            tb = t[b]
            gt = (kb > tb).astype(jnp.float32)
            eq = (kb == tb).astype(jnp.float32)
            cumc_eq = jax.lax.dot_general(eq, ucum, cd10,
                                          preferred_element_type=jnp.float32)
            req = jnp.sum(eq, axis=1, keepdims=True)
            req8 = jnp.concatenate([req, jnp.zeros((nr, 7), jnp.float32)], 1)
            peq = jax.lax.dot_general(lst, req8, cd10,
                                      preferred_element_type=jnp.float32)[:, 0:1]
            eqrank = cumc_eq + peq
            sel = gt + eq * (eqrank <= need[b]).astype(jnp.float32)
            cumc = jax.lax.dot_general(sel, ucum, cd10,
                                       preferred_element_type=jnp.float32)
            rrow = jax.lax.dot_general(ones8, sel, cd11,
                                       preferred_element_type=jnp.float32)
            prow = jax.lax.dot_general(rrow, ust, cd10,
                                       preferred_element_type=jnp.float32)[0:1]
            le = (prow <= pcol).astype(jnp.float32)
            rp = jnp.sum(le, axis=1, keepdims=True) - 1.0
            ohr = (rp == iota_rr).astype(jnp.float32)
            cumrow = jax.lax.dot_general(ohr, cumc, cd10,
                                         preferred_element_type=jnp.float32)
            prp = jnp.sum(ohr * prow, axis=1, keepdims=True)
            tp = pcol + 1.0 - prp
            cp = jnp.sum((cumrow < tp).astype(jnp.float32), axis=1,
                         keepdims=True)
            fcol = rp * float(nc) + cp
            scol = jnp.floor(fcol * (1.0 / MAXW))
            jcol = fcol - scol * float(MAXW)
            ecol = scol + float(MAXW - 1) - jcol
            sef_ref[b] = jnp.concatenate(
                [scol, ecol, fcol, jnp.zeros((kpad, 5), jnp.float32)], axis=1)
            x = seg_ref[b]
            w_col = ecol - scol
            oh_s = (scol == iota_row).astype(jnp.float32)
            oh_e = (ecol == iota_row).astype(jnp.float32)
            oh_w = (w_col == iota_wr).astype(jnp.float32)
            xg = jax.lax.dot_general(oh_s, x, cd10,
                                     preferred_element_type=jnp.float32)
            yg = jax.lax.dot_general(oh_e, x, cd10,
                                     preferred_element_type=jnp.float32)
            wg = jax.lax.dot_general(oh_w, wt_ref[...], cd10,
                                     preferred_element_type=jnp.float32)
            hpre = (jax.lax.dot_general(xg, aw, dims,
                                        preferred_element_type=jnp.float32)
                    + jax.lax.dot_general(yg, bw, dims,
                                          preferred_element_type=jnp.float32)
                    + jax.lax.dot_general(xg * yg, cw, dims,
                                          preferred_element_type=jnp.float32)
                    + jax.lax.dot_general(wg, dw, dims,
                                          preferred_element_type=jnp.float32)
                    + b1_ref[...])
            h = _gelu(hpre)
            emb_ref[b] = h
            sc = jax.lax.dot_general(h, w2p, dims,
                                     preferred_element_type=jnp.float32)
            pen = jnp.where(ecol >= seq_ref[b], -1e6, 0.0)
            scc_ref[b] = sc + b2_ref[0, 0] + pen
    return _select_kernel


def kernel(segment_embeddings, segment_mask, width_table, W1, b1, w2, b2):
    B, N, H = segment_embeddings.shape
    k = int(N * 0.4)
    kpad = k
    seq_lens = jnp.sum(segment_mask, axis=-1).reshape(B, 1, 1)
    b1r = b1.reshape(1, H)
    b2r = b2.reshape(1, 1)

    scores = pl.pallas_call(
        _score_kernel,
        grid=(B,),
        in_specs=[
            pl.BlockSpec((1, 1, 1), lambda b: (b, 0, 0)),
            pl.BlockSpec((1, N, H), lambda b: (b, 0, 0)),
            pl.BlockSpec((MAXW, WF), lambda b: (0, 0)),
            pl.BlockSpec((H, SPAN_DIM), lambda b: (0, 0)),
            pl.BlockSpec((1, H), lambda b: (0, 0)),
            pl.BlockSpec((1, H), lambda b: (0, 0)),
            pl.BlockSpec((1, 1), lambda b: (0, 0)),
        ],
        out_specs=pl.BlockSpec((1, 128, 128), lambda b: (b, 0, 0)),
        out_shape=jax.ShapeDtypeStruct((B, 128, 128), jnp.float32),
    )(seq_lens, segment_embeddings, width_table, W1, b1r, w2, b2r)

    sef, embs, scc = pl.pallas_call(
        _make_select_gather_kernel(B, N, k, kpad),
        grid=(1,),
        in_specs=[
            pl.BlockSpec((B, 128, 128), lambda i: (0, 0, 0)),
            pl.BlockSpec((B, 1, 1), lambda i: (0, 0, 0)),
            pl.BlockSpec((B, N, H), lambda i: (0, 0, 0)),
            pl.BlockSpec((MAXW, WF), lambda i: (0, 0)),
            pl.BlockSpec((H, SPAN_DIM), lambda i: (0, 0)),
            pl.BlockSpec((1, H), lambda i: (0, 0)),
            pl.BlockSpec((1, H), lambda i: (0, 0)),
            pl.BlockSpec((1, 1), lambda i: (0, 0)),
        ],
        out_specs=[
            pl.BlockSpec((B, kpad, 8), lambda i: (0, 0, 0)),
            pl.BlockSpec((B, kpad, H), lambda i: (0, 0, 0)),
            pl.BlockSpec((B, kpad, 8), lambda i: (0, 0, 0)),
        ],
        out_shape=[
            jax.ShapeDtypeStruct((B, kpad, 8), jnp.float32),
            jax.ShapeDtypeStruct((B, kpad, H), jnp.float32),
            jax.ShapeDtypeStruct((B, kpad, 8), jnp.float32),
        ],
    )(scores, seq_lens, segment_embeddings, width_table, W1, b1r, w2, b2r)

    top_embs = embs
    top_scores = scc[:, :, 0]
    top_spans = sef[:, :, 0:2].astype(jnp.int32)
    return top_embs, top_scores, top_spans


# speedup vs baseline: 14.2764x; 1.0534x over previous
import math

import jax
import jax.numpy as jnp
from jax.experimental import pallas as pl

HID = 256
MAXW = 16
WF = 20
SPAN_DIM = 3 * HID + WF
INT_MIN = -2147483648
INT_MAX = 2147483647


def _gelu(x):
    return 0.5 * x * (1.0 + jax.lax.erf(x * (1.0 / math.sqrt(2.0))))


def _score_kernel(seq_ref, seg_ref, wt_ref, w1_ref, b1_ref, w2_ref, b2_ref,
                  out_ref):
    n = seg_ref.shape[1]
    x = seg_ref[0]
    a = w1_ref[:, 0:HID]
    bm = w1_ref[:, HID:2 * HID]
    c = w1_ref[:, 2 * HID:3 * HID]
    d = w1_ref[:, 3 * HID:SPAN_DIM]
    dims = (((1,), (1,)), ((), ()))
    xa = jax.lax.dot_general(x, a, dims, preferred_element_type=jnp.float32)
    yb = jax.lax.dot_general(x, bm, dims, preferred_element_type=jnp.float32)
    wd = jax.lax.dot_general(wt_ref[...], d, dims,
                             preferred_element_type=jnp.float32) + b1_ref[...]
    seq_len = seq_ref[0, 0, 0]
    b2 = b2_ref[0, 0]
    w2p = jnp.concatenate(
        [w2_ref[...], jnp.zeros((7, HID), jnp.float32)], axis=0)
    s_iota = jax.lax.broadcasted_iota(jnp.int32, (n, 1), 0).astype(jnp.float32)
    cols = [None] * MAXW
    for w in range(MAXW):
        xw = x if w == 0 else jnp.roll(x, -w, axis=0)
        ybw = yb if w == 0 else jnp.roll(yb, -w, axis=0)
        p = x * xw
        hpre = xa + ybw + wd[w:w + 1, :] + jax.lax.dot_general(
            p, c, dims, preferred_element_type=jnp.float32)
        h = _gelu(hpre)
        sw = jax.lax.dot_general(h, w2p, dims,
                                 preferred_element_type=jnp.float32)[:, 0:1] + b2
        e = s_iota + float(w)
        sw = sw + jnp.where(e >= seq_len, -1e6, 0.0)
        cols[MAXW - 1 - w] = jnp.where(e >= float(n), -1e30, sw)
    out_ref[0] = jnp.concatenate(cols, axis=1)


def _to_key(x):
    x = jnp.where(x == 0.0, 0.0, x)
    b = jax.lax.bitcast_convert_type(x, jnp.int32)
    return jnp.where(b >= 0, b,
                     jnp.bitwise_xor(jnp.bitwise_not(b), jnp.int32(INT_MIN)))


def _make_select_gather_kernel(nbatch, n, k, kpad):
    def _select_kernel(sc128_ref, seq_ref, seg_ref, wt_ref, w1_ref,
                       b1_ref, w2_ref, b2_ref, sef_ref, emb_ref, scc_ref):
        keys_c = _to_key(sc128_ref[...])

        def count_ge(mid):
            ge = (keys_c >= mid).astype(jnp.int32)
            return jnp.sum(jnp.sum(ge, axis=2, keepdims=True), axis=1,
                           keepdims=True)

        zero = jnp.zeros((nbatch, 1, 1), jnp.int32)
        c0 = count_ge(zero)
        lo = jnp.where(c0 >= k, 0, INT_MIN)
        hi = jnp.where(c0 >= k, INT_MAX, -1)

        def body4(_, carry):
            lo, hi = carry
            d = hi - lo
            d2 = jax.lax.shift_right_arithmetic(d, 2)
            r = d & 3
            m1 = lo + d2 + (r > 0).astype(jnp.int32)
            m2 = lo + 2 * d2 + jax.lax.shift_right_arithmetic(r + 1, 1)
            m3 = lo + 3 * d2 + jax.lax.shift_right_arithmetic(3 * r + 3, 2)
            g1 = count_ge(m1) >= k
            g2 = count_ge(m2) >= k
            g3 = count_ge(m3) >= k
            nlo = jnp.where(g3, m3, jnp.where(g2, m2, jnp.where(g1, m1, lo)))
            nhi = jnp.where(g3, hi,
                            jnp.where(g2, m3 - 1,
                                      jnp.where(g1, m2 - 1, m1 - 1)))
            return nlo, nhi

        def body2(_, carry):
            lo, hi = carry
            d = hi - lo
            mid = lo + jax.lax.shift_right_arithmetic(d, 1) + (d & 1)
            ge = count_ge(mid) >= k
            return jnp.where(ge, mid, lo), jnp.where(ge, hi, mid - 1)

        lo, hi = jax.lax.fori_loop(0, 16, body4, (lo, hi))
        lo, hi = jax.lax.fori_loop(0, 2, body2, (lo, hi))
        t = lo
        m = jnp.sum(jnp.sum((keys_c > t).astype(jnp.int32), axis=2,
                            keepdims=True), axis=1, keepdims=True)
        need = (k - m).astype(jnp.float32)

        nr = sc128_ref.shape[1]
        nc = sc128_ref.shape[2]
        iota_a = jax.lax.broadcasted_iota(jnp.int32, (nc, nc), 0)
        iota_b = jax.lax.broadcasted_iota(jnp.int32, (nc, nc), 1)
        ucum = (iota_a <= iota_b).astype(jnp.float32)
        lst = (iota_b < iota_a).astype(jnp.float32)
        ust = (iota_a < iota_b).astype(jnp.float32)
        ones8 = jnp.ones((8, nc), jnp.float32)
        pcol = jax.lax.broadcasted_iota(jnp.int32, (kpad, 1), 0).astype(
            jnp.float32)
        iota_rr = jax.lax.broadcasted_iota(jnp.int32, (1, nr), 1).astype(
            jnp.float32)
        iota_row = jax.lax.broadcasted_iota(jnp.int32, (1, n), 1).astype(
            jnp.float32)
        cd10 = (((1,), (0,)), ((), ()))
        cd11 = (((1,), (1,)), ((), ()))
        dims = (((1,), (1,)), ((), ()))
        iota_wr = jax.lax.broadcasted_iota(jnp.int32, (1, MAXW), 1).astype(
            jnp.float32)
        aw = w1_ref[:, 0:HID]
        bw = w1_ref[:, HID:2 * HID]
        cw = w1_ref[:, 2 * HID:3 * HID]
        dw = w1_ref[:, 3 * HID:SPAN_DIM]
        w2p = jnp.concatenate(
            [w2_ref[...], jnp.zeros((7, HID), jnp.float32)], axis=0)
        for b in range(nbatch):
            kb = keys_c[b]
            tb = t[b]
            gt = (kb > tb).astype(jnp.float32)
            eq = (kb == tb).astype(jnp.float32)
            cumc_eq = jax.lax.dot_general(eq, ucum, cd10,
                                          preferred_element_type=jnp.float32)
            req = jnp.sum(eq, axis=1, keepdims=True)
            req8 = jnp.concatenate([req, jnp.zeros((nr, 7), jnp.float32)], 1)
            peq = jax.lax.dot_general(lst, req8, cd10,
                                      preferred_element_type=jnp.float32)[:, 0:1]
            eqrank = cumc_eq + peq
            sel = gt + eq * (eqrank <= need[b]).astype(jnp.float32)
            cumc = jax.lax.dot_general(sel, ucum, cd10,
                                       preferred_element_type=jnp.float32)
            rrow = jax.lax.dot_general(ones8, sel, cd11,
                                       preferred_element_type=jnp.float32)
            prow = jax.lax.dot_general(rrow, ust, cd10,
                                       preferred_element_type=jnp.float32)[0:1]
            le = (prow <= pcol).astype(jnp.float32)
            rp = jnp.sum(le, axis=1, keepdims=True) - 1.0
            ohr = (rp == iota_rr).astype(jnp.float32)
            cumrow = jax.lax.dot_general(ohr, cumc, cd10,
                                         preferred_element_type=jnp.float32)
            prp = jnp.sum(ohr * prow, axis=1, keepdims=True)
            tp = pcol + 1.0 - prp
            cp = jnp.sum((cumrow < tp).astype(jnp.float32), axis=1,
                         keepdims=True)
            fcol = rp * float(nc) + cp
            scol = jnp.floor(fcol * (1.0 / MAXW))
            jcol = fcol - scol * float(MAXW)
            ecol = scol + float(MAXW - 1) - jcol
            sef_ref[b] = jnp.concatenate(
                [scol, ecol, fcol, jnp.zeros((kpad, 5), jnp.float32)], axis=1)
            x = seg_ref[b]
            w_col = ecol - scol
            oh_s = (scol == iota_row).astype(jnp.float32)
            oh_e = (ecol == iota_row).astype(jnp.float32)
            oh_w = (w_col == iota_wr).astype(jnp.float32)
            xg = jax.lax.dot_general(oh_s, x, cd10,
                                     preferred_element_type=jnp.float32)
            yg = jax.lax.dot_general(oh_e, x, cd10,
                                     preferred_element_type=jnp.float32)
            wg = jax.lax.dot_general(oh_w, wt_ref[...], cd10,
                                     preferred_element_type=jnp.float32)
            hpre = (jax.lax.dot_general(xg, aw, dims,
                                        preferred_element_type=jnp.float32)
                    + jax.lax.dot_general(yg, bw, dims,
                                          preferred_element_type=jnp.float32)
                    + jax.lax.dot_general(xg * yg, cw, dims,
                                          preferred_element_type=jnp.float32)
                    + jax.lax.dot_general(wg, dw, dims,
                                          preferred_element_type=jnp.float32)
                    + b1_ref[...])
            h = _gelu(hpre)
            emb_ref[b] = h
            sc = jax.lax.dot_general(h, w2p, dims,
                                     preferred_element_type=jnp.float32)
            pen = jnp.where(ecol >= seq_ref[b], -1e6, 0.0)
            scc_ref[b] = sc + b2_ref[0, 0] + pen
    return _select_kernel


def kernel(segment_embeddings, segment_mask, width_table, W1, b1, w2, b2):
    B, N, H = segment_embeddings.shape
    k = int(N * 0.4)
    kpad = ((k + 7) // 8) * 8
    seq_lens = jnp.sum(segment_mask, axis=-1).reshape(B, 1, 1)
    b1r = b1.reshape(1, H)
    b2r = b2.reshape(1, 1)

    scores = pl.pallas_call(
        _score_kernel,
        grid=(B,),
        in_specs=[
            pl.BlockSpec((1, 1, 1), lambda b: (b, 0, 0)),
            pl.BlockSpec((1, N, H), lambda b: (b, 0, 0)),
            pl.BlockSpec((MAXW, WF), lambda b: (0, 0)),
            pl.BlockSpec((H, SPAN_DIM), lambda b: (0, 0)),
            pl.BlockSpec((1, H), lambda b: (0, 0)),
            pl.BlockSpec((1, H), lambda b: (0, 0)),
            pl.BlockSpec((1, 1), lambda b: (0, 0)),
        ],
        out_specs=pl.BlockSpec((1, N, MAXW), lambda b: (b, 0, 0)),
        out_shape=jax.ShapeDtypeStruct((B, N, MAXW), jnp.float32),
    )(seq_lens, segment_embeddings, width_table, W1, b1r, w2, b2r)

    sc128 = scores.reshape(B, 128, (N * MAXW) // 128)

    sef, embs, scc = pl.pallas_call(
        _make_select_gather_kernel(B, N, k, kpad),
        grid=(1,),
        in_specs=[
            pl.BlockSpec((B, 128, (N * MAXW) // 128), lambda i: (0, 0, 0)),
            pl.BlockSpec((B, 1, 1), lambda i: (0, 0, 0)),
            pl.BlockSpec((B, N, H), lambda i: (0, 0, 0)),
            pl.BlockSpec((MAXW, WF), lambda i: (0, 0)),
            pl.BlockSpec((H, SPAN_DIM), lambda i: (0, 0)),
            pl.BlockSpec((1, H), lambda i: (0, 0)),
            pl.BlockSpec((1, H), lambda i: (0, 0)),
            pl.BlockSpec((1, 1), lambda i: (0, 0)),
        ],
        out_specs=[
            pl.BlockSpec((B, kpad, 8), lambda i: (0, 0, 0)),
            pl.BlockSpec((B, kpad, H), lambda i: (0, 0, 0)),
            pl.BlockSpec((B, kpad, 8), lambda i: (0, 0, 0)),
        ],
        out_shape=[
            jax.ShapeDtypeStruct((B, kpad, 8), jnp.float32),
            jax.ShapeDtypeStruct((B, kpad, H), jnp.float32),
            jax.ShapeDtypeStruct((B, kpad, 8), jnp.float32),
        ],
    )(sc128, seq_lens, segment_embeddings, width_table, W1, b1r, w2, b2r)

    top_embs = embs[:, :k, :]
    top_scores = scc[:, :k, 0]
    top_spans = sef[:, :k, 0:2].astype(jnp.int32)
    return top_embs, top_scores, top_spans
